# Initial kernel scaffold; baseline (speedup 1.0000x reference)
import jax
import jax.numpy as jnp
from jax.experimental import pallas as pl


def kernel(x0, W):
    raise NotImplementedError("write your pallas kernel here")


# speedup vs baseline: 9.9327x; 9.9327x over previous
import functools

import jax
import jax.numpy as jnp
from jax import lax
from jax.experimental import pallas as pl
from jax.experimental.pallas import tpu as pltpu
from jax.experimental.pallas import tpu_sc as plsc

T = 512
K = 8192
D = 400
G = 64
PG = 128
S = 8
COMMIT = 0.25

RT = 64
NSTEP = T // RT
NW = 32
BPW = (T * S) // NW

_HI = lax.Precision.HIGHEST


DPAD = 512
KT = K // NSTEP


def _stage1_kernel(x_ref, w_ref, enc_ref, idx_ref, prob_ref, ent_ref,
                   wpad_ref, hist_ref):
    step = pl.program_id(0)
    x = x_ref[...]
    w = w_ref[...]

    wq = w_ref[pl.ds(step * KT, KT), :]
    wpad_ref[...] = jnp.concatenate(
        [wq, jnp.zeros((KT, DPAD - D), jnp.float32)], axis=1)

    xx = jnp.sum(x * x, axis=1, keepdims=True)
    ww = jnp.sum(w * w, axis=1)
    xw = lax.dot_general(
        x.astype(jnp.bfloat16), w.astype(jnp.bfloat16),
        (((1,), (1,)), ((), ())),
        preferred_element_type=jnp.float32)
    d = xx + ww[None, :] - 2.0 * xw
    enc = 1.0 / d
    enc_ref[...] = enc

    a_row = lax.broadcasted_iota(jnp.int32, (K, G), 0) // PG
    a_col = lax.broadcasted_iota(jnp.int32, (K, G), 1)
    amat = (a_row == a_col).astype(jnp.float32)
    d_group = lax.dot_general(
        d, amat, (((1,), (0,)), ((), ())),
        precision=_HI, preferred_element_type=jnp.float32)
    gmin = jnp.min(d_group, axis=1, keepdims=True)
    giota = lax.broadcasted_iota(jnp.int32, (RT, G), 1)
    g = jnp.min(jnp.where(d_group == gmin, giota, G), axis=1, keepdims=True)

    col = lax.broadcasted_iota(jnp.int32, (RT, K), 1)
    masked_e = jnp.where(col // PG == g, enc, 0.0)
    b_row = lax.broadcasted_iota(jnp.int32, (K, PG), 0) % PG
    b_col = lax.broadcasted_iota(jnp.int32, (K, PG), 1)
    bmat = (b_row == b_col).astype(jnp.float32)
    e_win = lax.dot_general(
        masked_e, bmat, (((1,), (0,)), ((), ())),
        precision=_HI, preferred_element_type=jnp.float32)

    lane = lax.broadcasted_iota(jnp.int32, (RT, PG), 1)
    cur = e_win
    ps = []
    js = []
    for _ in range(S):
        m = jnp.max(cur, axis=1, keepdims=True)
        j = jnp.min(jnp.where(cur == m, lane, PG), axis=1, keepdims=True)
        ps.append(m)
        js.append(j)
        cur = jnp.where(lane == j, -1.0, cur)

    denom = jnp.maximum(
        ps[0] + ps[1] + ps[2] + ps[3] + ps[4] + ps[5] + ps[6] + ps[7], 1e-12)
    prob_ref[...] = jnp.concatenate(ps, axis=1) / denom
    idx_ref[...] = g * PG + jnp.concatenate(js, axis=1)

    @pl.when(step == 0)
    def _init():
        hist_ref[...] = jnp.zeros((1, K), jnp.float32)

    idx0 = g * PG + js[0]
    hit = (col == idx0).astype(jnp.float32)
    hist_ref[...] += jnp.sum(hit, axis=0, keepdims=True)

    @pl.when(step == NSTEP - 1)
    def _finish():
        hist = hist_ref[...]
        prob = hist * (1.0 / T)
        plogp = jnp.where(
            hist > 0, prob * jnp.log(jnp.where(hist > 0, prob, 1.0)), 0.0)
        ent_ref[...] = -jnp.sum(plogp, axis=(0, 1), keepdims=True)


@functools.cache
def _sc_gather_fn():
    @functools.partial(
        pl.kernel,
        out_type=jax.ShapeDtypeStruct((T * S, DPAD), jnp.float32),
        mesh=plsc.VectorSubcoreMesh(core_axis_name="c", subcore_axis_name="s"),
        scratch_types=[
            pltpu.VMEM((BPW,), jnp.int32),
            pltpu.VMEM((BPW, DPAD), jnp.float32),
            pltpu.SemaphoreType.DMA,
        ],
    )
    def _sc_gather(w_hbm, idx_hbm, out_hbm, idx_v, rows_v, sem):
        wid = lax.axis_index("s") * 2 + lax.axis_index("c")
        base = wid * BPW
        pltpu.sync_copy(idx_hbm.at[pl.ds(base, BPW)], idx_v)
        pltpu.async_copy(w_hbm.at[idx_v], rows_v, sem).wait()
        pltpu.sync_copy(rows_v, out_hbm.at[pl.ds(base, BPW)])

    return _sc_gather


def _stage3_kernel(x_ref, gath_ref, prob_ref, out_ref, loss_ref):
    x = x_ref[...]
    pr = prob_ref[...]
    g3 = gath_ref[...].reshape(T, S, DPAD)
    out = g3[:, 0, :D] * pr[:, 0:1]
    for s in range(1, S):
        out = out + g3[:, s, :D] * pr[:, s:s + 1]
    out_ref[...] = (out - x) + x
    diff = x - out
    mse = jnp.sum(diff * diff, axis=(0, 1), keepdims=True) * (1.0 / (T * D))
    loss_ref[...] = mse + COMMIT * mse


def kernel(x0, W):
    x1 = x0.reshape(T, D)

    enc, idx8, prob8, ent, wpad = pl.pallas_call(
        _stage1_kernel,
        grid=(NSTEP,),
        in_specs=[
            pl.BlockSpec((RT, D), lambda i: (i, 0)),
            pl.BlockSpec((K, D), lambda i: (0, 0)),
        ],
        out_specs=[
            pl.BlockSpec((RT, K), lambda i: (i, 0)),
            pl.BlockSpec((RT, S), lambda i: (i, 0)),
            pl.BlockSpec((RT, S), lambda i: (i, 0)),
            pl.BlockSpec((1, 1), lambda i: (0, 0)),
            pl.BlockSpec((KT, DPAD), lambda i: (i, 0)),
        ],
        out_shape=(
            jax.ShapeDtypeStruct((T, K), jnp.float32),
            jax.ShapeDtypeStruct((T, S), jnp.int32),
            jax.ShapeDtypeStruct((T, S), jnp.float32),
            jax.ShapeDtypeStruct((1, 1), jnp.float32),
            jax.ShapeDtypeStruct((K, DPAD), jnp.float32),
        ),
        scratch_shapes=[pltpu.VMEM((1, K), jnp.float32)],
        compiler_params=pltpu.CompilerParams(
            dimension_semantics=("arbitrary",),
            vmem_limit_bytes=100 * 1024 * 1024,
        ),
    )(x1, W)

    gathered = _sc_gather_fn()(wpad, idx8.reshape(T * S))

    out_flat, loss = pl.pallas_call(
        _stage3_kernel,
        out_shape=(
            jax.ShapeDtypeStruct((T, D), jnp.float32),
            jax.ShapeDtypeStruct((1, 1), jnp.float32),
        ),
    )(x1, gathered, prob8)

    return (loss[0, 0], out_flat.reshape(x0.shape), ent[0, 0], enc[None])


# speedup vs baseline: 10.7272x; 1.0800x over previous
import functools

import jax
import jax.numpy as jnp
from jax import lax
from jax.experimental import pallas as pl
from jax.experimental.pallas import tpu as pltpu
from jax.experimental.pallas import tpu_sc as plsc

T = 512
K = 8192
D = 400
G = 64
PG = 128
S = 8
COMMIT = 0.25

RT = 64
NSTEP = T // RT
NW = 32
BPW = (T * S) // NW

_HI = lax.Precision.HIGHEST


DPAD = 512
KT = K // NSTEP


def _stage1_kernel(x_ref, w_ref, enc_ref, idx_ref, prob_ref, ent_ref,
                   wpad_ref, hist_ref, ww_ref, amat_ref):
    step = pl.program_id(0)
    x = x_ref[...]
    w = w_ref[...]

    wq = w_ref[pl.ds(step * KT, KT), :]
    wpad_ref[...] = jnp.concatenate(
        [wq, jnp.zeros((KT, DPAD - D), jnp.float32)], axis=1)

    @pl.when(step == 0)
    def _precompute():
        ww_ref[...] = jnp.sum(w * w, axis=1)[None, :]
        a_row = lax.broadcasted_iota(jnp.int32, (K, G), 0) // PG
        a_col = lax.broadcasted_iota(jnp.int32, (K, G), 1)
        amat_ref[...] = (a_row == a_col).astype(jnp.float32)

    xx = jnp.sum(x * x, axis=1, keepdims=True)
    xw = lax.dot_general(
        x.astype(jnp.bfloat16), w.astype(jnp.bfloat16),
        (((1,), (1,)), ((), ())),
        preferred_element_type=jnp.float32)
    d = xx + ww_ref[...] - 2.0 * xw
    enc = 1.0 / d
    enc_ref[...] = enc

    d_group = lax.dot_general(
        d, amat_ref[...], (((1,), (0,)), ((), ())),
        precision=_HI, preferred_element_type=jnp.float32)
    gmin = jnp.min(d_group, axis=1, keepdims=True)
    giota = lax.broadcasted_iota(jnp.int32, (RT, G), 1)
    g = jnp.min(jnp.where(d_group == gmin, giota, G), axis=1, keepdims=True)

    cur = enc
    for b in (32, 16, 8, 4, 2, 1):
        half = cur.shape[1] // 2
        cur = jnp.where((g & b) != 0, cur[:, half:], cur[:, :half])
    e_win = cur

    lane = lax.broadcasted_iota(jnp.int32, (RT, PG), 1)
    cur = e_win
    ps = []
    js = []
    for _ in range(S):
        m = jnp.max(cur, axis=1, keepdims=True)
        j = jnp.min(jnp.where(cur == m, lane, PG), axis=1, keepdims=True)
        ps.append(m)
        js.append(j)
        cur = jnp.where(lane == j, -1.0, cur)

    denom = jnp.maximum(
        ps[0] + ps[1] + ps[2] + ps[3] + ps[4] + ps[5] + ps[6] + ps[7], 1e-12)
    prob_ref[...] = jnp.concatenate(ps, axis=1) / denom
    idx_ref[...] = g * PG + jnp.concatenate(js, axis=1)

    @pl.when(step == 0)
    def _init():
        hist_ref[...] = jnp.zeros((1, K), jnp.float32)

    idx0 = g * PG + js[0]
    col = lax.broadcasted_iota(jnp.int32, (RT, K), 1)
    hit = (col == idx0).astype(jnp.float32)
    hist_ref[...] += jnp.sum(hit, axis=0, keepdims=True)

    @pl.when(step == NSTEP - 1)
    def _finish():
        hist = hist_ref[...]
        prob = hist * (1.0 / T)
        plogp = jnp.where(
            hist > 0, prob * jnp.log(jnp.where(hist > 0, prob, 1.0)), 0.0)
        ent_ref[...] = -jnp.sum(plogp, axis=(0, 1), keepdims=True)


@functools.cache
def _sc_gather_fn():
    @functools.partial(
        pl.kernel,
        out_type=jax.ShapeDtypeStruct((T * S, DPAD), jnp.float32),
        mesh=plsc.VectorSubcoreMesh(core_axis_name="c", subcore_axis_name="s"),
        scratch_types=[
            pltpu.VMEM((BPW,), jnp.int32),
            pltpu.VMEM((BPW, DPAD), jnp.float32),
            pltpu.SemaphoreType.DMA,
        ],
    )
    def _sc_gather(w_hbm, idx_hbm, out_hbm, idx_v, rows_v, sem):
        wid = lax.axis_index("s") * 2 + lax.axis_index("c")
        base = wid * BPW
        pltpu.sync_copy(idx_hbm.at[pl.ds(base, BPW)], idx_v)
        pltpu.async_copy(w_hbm.at[idx_v], rows_v, sem).wait()
        pltpu.sync_copy(rows_v, out_hbm.at[pl.ds(base, BPW)])

    return _sc_gather


def _stage3_kernel(x_ref, gath_ref, prob_ref, out_ref, loss_ref):
    x = x_ref[...]
    pr = prob_ref[...]
    g3 = gath_ref[...].reshape(T, S, DPAD)
    out = g3[:, 0, :D] * pr[:, 0:1]
    for s in range(1, S):
        out = out + g3[:, s, :D] * pr[:, s:s + 1]
    out_ref[...] = (out - x) + x
    diff = x - out
    mse = jnp.sum(diff * diff, axis=(0, 1), keepdims=True) * (1.0 / (T * D))
    loss_ref[...] = mse + COMMIT * mse


def kernel(x0, W):
    x1 = x0.reshape(T, D)

    enc, idx8, prob8, ent, wpad = pl.pallas_call(
        _stage1_kernel,
        grid=(NSTEP,),
        in_specs=[
            pl.BlockSpec((RT, D), lambda i: (i, 0)),
            pl.BlockSpec((K, D), lambda i: (0, 0)),
        ],
        out_specs=[
            pl.BlockSpec((RT, K), lambda i: (i, 0)),
            pl.BlockSpec((RT, S), lambda i: (i, 0)),
            pl.BlockSpec((RT, S), lambda i: (i, 0)),
            pl.BlockSpec((1, 1), lambda i: (0, 0)),
            pl.BlockSpec((KT, DPAD), lambda i: (i, 0)),
        ],
        out_shape=(
            jax.ShapeDtypeStruct((T, K), jnp.float32),
            jax.ShapeDtypeStruct((T, S), jnp.int32),
            jax.ShapeDtypeStruct((T, S), jnp.float32),
            jax.ShapeDtypeStruct((1, 1), jnp.float32),
            jax.ShapeDtypeStruct((K, DPAD), jnp.float32),
        ),
        scratch_shapes=[
            pltpu.VMEM((1, K), jnp.float32),
            pltpu.VMEM((1, K), jnp.float32),
            pltpu.VMEM((K, G), jnp.float32),
        ],
        compiler_params=pltpu.CompilerParams(
            dimension_semantics=("arbitrary",),
            vmem_limit_bytes=100 * 1024 * 1024,
        ),
    )(x1, W)

    gathered = _sc_gather_fn()(wpad, idx8.reshape(T * S))

    out_flat, loss = pl.pallas_call(
        _stage3_kernel,
        out_shape=(
            jax.ShapeDtypeStruct((T, D), jnp.float32),
            jax.ShapeDtypeStruct((1, 1), jnp.float32),
        ),
    )(x1, gathered, prob8)

    return (loss[0, 0], out_flat.reshape(x0.shape), ent[0, 0], enc[None])


# speedup vs baseline: 10.8589x; 1.0123x over previous
import functools

import jax
import jax.numpy as jnp
from jax import lax
from jax.experimental import pallas as pl
from jax.experimental.pallas import tpu as pltpu
from jax.experimental.pallas import tpu_sc as plsc

T = 512
K = 8192
D = 400
G = 64
PG = 128
S = 8
COMMIT = 0.25

RT = 64
NSTEP = T // RT
NW = 32
BPW = (T * S) // NW

_HI = lax.Precision.HIGHEST


DPAD = 512
KT = K // NSTEP


def _stage1_kernel(x_ref, w_ref, enc_ref, idx_ref, prob_ref,
                   wpad_ref, ww_ref, amat_ref, wb_ref):
    step = pl.program_id(0)
    x = x_ref[...]
    w = w_ref[...]

    wq = w_ref[pl.ds(step * KT, KT), :]
    wpad_ref[...] = jnp.concatenate(
        [wq, jnp.zeros((KT, DPAD - D), jnp.float32)], axis=1)

    @pl.when(step == 0)
    def _precompute():
        ww_ref[...] = jnp.sum(w * w, axis=1)[None, :]
        a_row = lax.broadcasted_iota(jnp.int32, (K, G), 0) // PG
        a_col = lax.broadcasted_iota(jnp.int32, (K, G), 1)
        amat_ref[...] = (a_row == a_col).astype(jnp.float32)
        wb_ref[...] = w.astype(jnp.bfloat16)

    xx = jnp.sum(x * x, axis=1, keepdims=True)
    xw = lax.dot_general(
        x.astype(jnp.bfloat16), wb_ref[...],
        (((1,), (1,)), ((), ())),
        preferred_element_type=jnp.float32)
    d = xx + ww_ref[...] - 2.0 * xw
    enc = 1.0 / d
    enc_ref[...] = enc

    d_group = lax.dot_general(
        d, amat_ref[...], (((1,), (0,)), ((), ())),
        precision=_HI, preferred_element_type=jnp.float32)
    gmin = jnp.min(d_group, axis=1, keepdims=True)
    giota = lax.broadcasted_iota(jnp.int32, (RT, G), 1)
    g = jnp.min(jnp.where(d_group == gmin, giota, G), axis=1, keepdims=True)

    cur = enc
    for b in (32, 16, 8, 4, 2, 1):
        half = cur.shape[1] // 2
        cur = jnp.where((g & b) != 0, cur[:, half:], cur[:, :half])
    e_win = cur

    lane = lax.broadcasted_iota(jnp.int32, (RT, PG), 1)
    cur = e_win
    ps = []
    js = []
    for _ in range(S):
        m = jnp.max(cur, axis=1, keepdims=True)
        j = jnp.min(jnp.where(cur == m, lane, PG), axis=1, keepdims=True)
        ps.append(m)
        js.append(j)
        cur = jnp.where(lane == j, -1.0, cur)

    denom = jnp.maximum(
        ps[0] + ps[1] + ps[2] + ps[3] + ps[4] + ps[5] + ps[6] + ps[7], 1e-12)
    prob_ref[...] = jnp.concatenate(ps, axis=1) / denom
    idx_ref[...] = g * PG + jnp.concatenate(js, axis=1)


@functools.cache
def _sc_gather_fn():
    @functools.partial(
        pl.kernel,
        out_type=jax.ShapeDtypeStruct((T * S, DPAD), jnp.float32),
        mesh=plsc.VectorSubcoreMesh(core_axis_name="c", subcore_axis_name="s"),
        scratch_types=[
            pltpu.VMEM((BPW,), jnp.int32),
            pltpu.VMEM((BPW, DPAD), jnp.float32),
            pltpu.SemaphoreType.DMA,
        ],
    )
    def _sc_gather(w_hbm, idx_hbm, out_hbm, idx_v, rows_v, sem):
        wid = lax.axis_index("s") * 2 + lax.axis_index("c")
        base = wid * BPW
        pltpu.sync_copy(idx_hbm.at[pl.ds(base, BPW)], idx_v)
        pltpu.async_copy(w_hbm.at[idx_v], rows_v, sem).wait()
        pltpu.sync_copy(rows_v, out_hbm.at[pl.ds(base, BPW)])

    return _sc_gather


def _stage3_kernel(x_ref, gath_ref, prob_ref, idx_ref, out_ref, loss_ref,
                   ent_ref):
    x = x_ref[...]
    pr = prob_ref[...]
    g3 = gath_ref[...].reshape(T, S, DPAD)
    out = g3[:, 0, :D] * pr[:, 0:1]
    for s in range(1, S):
        out = out + g3[:, s, :D] * pr[:, s:s + 1]
    out_ref[...] = (out - x) + x
    diff = x - out
    mse = jnp.sum(diff * diff, axis=(0, 1), keepdims=True) * (1.0 / (T * D))
    loss_ref[...] = mse + COMMIT * mse

    idx0 = idx_ref[...][:, 0:1]
    eq = (idx0 == idx0.reshape(1, T)).astype(jnp.float32)
    n = jnp.sum(eq, axis=1, keepdims=True)
    ent_ref[...] = jnp.sum(
        jnp.log(n * (1.0 / T)), axis=(0, 1), keepdims=True) * (-1.0 / T)


def kernel(x0, W):
    x1 = x0.reshape(T, D)

    enc, idx8, prob8, wpad = pl.pallas_call(
        _stage1_kernel,
        grid=(NSTEP,),
        in_specs=[
            pl.BlockSpec((RT, D), lambda i: (i, 0)),
            pl.BlockSpec((K, D), lambda i: (0, 0)),
        ],
        out_specs=[
            pl.BlockSpec((RT, K), lambda i: (i, 0)),
            pl.BlockSpec((RT, S), lambda i: (i, 0)),
            pl.BlockSpec((RT, S), lambda i: (i, 0)),
            pl.BlockSpec((KT, DPAD), lambda i: (i, 0)),
        ],
        out_shape=(
            jax.ShapeDtypeStruct((T, K), jnp.float32),
            jax.ShapeDtypeStruct((T, S), jnp.int32),
            jax.ShapeDtypeStruct((T, S), jnp.float32),
            jax.ShapeDtypeStruct((K, DPAD), jnp.float32),
        ),
        scratch_shapes=[
            pltpu.VMEM((1, K), jnp.float32),
            pltpu.VMEM((K, G), jnp.float32),
            pltpu.VMEM((K, D), jnp.bfloat16),
        ],
        compiler_params=pltpu.CompilerParams(
            dimension_semantics=("arbitrary",),
            vmem_limit_bytes=100 * 1024 * 1024,
        ),
    )(x1, W)

    gathered = _sc_gather_fn()(wpad, idx8.reshape(T * S))

    out_flat, loss, ent = pl.pallas_call(
        _stage3_kernel,
        out_shape=(
            jax.ShapeDtypeStruct((T, D), jnp.float32),
            jax.ShapeDtypeStruct((1, 1), jnp.float32),
            jax.ShapeDtypeStruct((1, 1), jnp.float32),
        ),
    )(x1, gathered, prob8, idx8)

    return (loss[0, 0], out_flat.reshape(x0.shape), ent[0, 0], enc[None])


# speedup vs baseline: 14.0584x; 1.2946x over previous
import functools

import jax
import jax.numpy as jnp
from jax import lax
from jax.experimental import pallas as pl
from jax.experimental.pallas import tpu as pltpu
from jax.experimental.pallas import tpu_sc as plsc

T = 512
K = 8192
D = 400
G = 64
PG = 128
S = 8
COMMIT = 0.25

RT = 128
NSTEP = T // RT
NW = 32
BPW = (T * S) // NW

_HI = lax.Precision.HIGHEST


DPAD = 512
KT = K // NSTEP


def _stage1_kernel(x_ref, w_ref, enc_ref, idx_ref, prob_ref,
                   wpad_ref, ww_ref, amat_ref, wb_ref):
    step = pl.program_id(0)
    x = x_ref[...]
    w = w_ref[...]

    wpad_ref[:, :D] = w_ref[pl.ds(step * KT, KT), :]

    @pl.when(step == 0)
    def _precompute():
        ww_ref[...] = jnp.sum(w * w, axis=1)[None, :]
        a_row = lax.broadcasted_iota(jnp.int32, (K, G), 0) // PG
        a_col = lax.broadcasted_iota(jnp.int32, (K, G), 1)
        amat_ref[...] = (a_row == a_col).astype(jnp.float32)
        wb_ref[...] = w.astype(jnp.bfloat16).T

    xx = jnp.sum(x * x, axis=1, keepdims=True)
    xw = lax.dot_general(
        x.astype(jnp.bfloat16), wb_ref[...],
        (((1,), (0,)), ((), ())),
        preferred_element_type=jnp.float32)
    d = xx + ww_ref[...] - 2.0 * xw
    enc = 1.0 / d
    enc_ref[...] = enc

    d_group = lax.dot_general(
        d, amat_ref[...], (((1,), (0,)), ((), ())),
        precision=_HI, preferred_element_type=jnp.float32)
    gmin = jnp.min(d_group, axis=1, keepdims=True)
    giota = lax.broadcasted_iota(jnp.int32, (RT, G), 1)
    g = jnp.min(jnp.where(d_group == gmin, giota, G), axis=1, keepdims=True)

    cur = enc
    for b in (32, 16, 8, 4, 2, 1):
        half = cur.shape[1] // 2
        cur = jnp.where((g & b) != 0, cur[:, half:], cur[:, :half])
    e_win = cur

    lane = lax.broadcasted_iota(jnp.int32, (RT, PG), 1)
    cur = e_win
    ps = []
    js = []
    for _ in range(S):
        m = jnp.max(cur, axis=1, keepdims=True)
        j = jnp.min(jnp.where(cur == m, lane, PG), axis=1, keepdims=True)
        ps.append(m)
        js.append(j)
        cur = jnp.where(lane == j, -1.0, cur)

    denom = jnp.maximum(
        ps[0] + ps[1] + ps[2] + ps[3] + ps[4] + ps[5] + ps[6] + ps[7], 1e-12)
    prob_ref[...] = jnp.concatenate(ps, axis=1) / denom
    idx_ref[...] = g * PG + jnp.concatenate(js, axis=1)


@functools.cache
def _sc_gather_fn():
    @functools.partial(
        pl.kernel,
        out_type=jax.ShapeDtypeStruct((T * S, DPAD), jnp.float32),
        mesh=plsc.VectorSubcoreMesh(core_axis_name="c", subcore_axis_name="s"),
        scratch_types=[
            pltpu.VMEM((BPW,), jnp.int32),
            pltpu.VMEM((BPW, DPAD), jnp.float32),
            pltpu.SemaphoreType.DMA,
        ],
    )
    def _sc_gather(w_hbm, idx_hbm, out_hbm, idx_v, rows_v, sem):
        wid = lax.axis_index("s") * 2 + lax.axis_index("c")
        base = wid * BPW
        pltpu.sync_copy(idx_hbm.at[pl.ds(base, BPW)], idx_v)
        pltpu.async_copy(w_hbm.at[idx_v], rows_v, sem).wait()
        pltpu.sync_copy(rows_v, out_hbm.at[pl.ds(base, BPW)])

    return _sc_gather


def _stage3_kernel(x_ref, gath_ref, prob_ref, idx_ref, out_ref, loss_ref,
                   ent_ref):
    x = x_ref[...]
    pr = prob_ref[...]
    g3 = gath_ref[...].reshape(T, S, DPAD)
    out = g3[:, 0, :D] * pr[:, 0:1]
    for s in range(1, S):
        out = out + g3[:, s, :D] * pr[:, s:s + 1]
    out_ref[...] = (out - x) + x
    diff = x - out
    mse = jnp.sum(diff * diff, axis=(0, 1), keepdims=True) * (1.0 / (T * D))
    loss_ref[...] = mse + COMMIT * mse

    idx0 = idx_ref[...][:, 0:1]
    eq = (idx0 == idx0.reshape(1, T)).astype(jnp.float32)
    n = jnp.sum(eq, axis=1, keepdims=True)
    ent_ref[...] = jnp.sum(
        jnp.log(n * (1.0 / T)), axis=(0, 1), keepdims=True) * (-1.0 / T)


def kernel(x0, W):
    x1 = x0.reshape(T, D)

    enc, idx8, prob8, wpad = pl.pallas_call(
        _stage1_kernel,
        grid=(NSTEP,),
        in_specs=[
            pl.BlockSpec((RT, D), lambda i: (i, 0)),
            pl.BlockSpec((K, D), lambda i: (0, 0)),
        ],
        out_specs=[
            pl.BlockSpec((RT, K), lambda i: (i, 0)),
            pl.BlockSpec((RT, S), lambda i: (i, 0)),
            pl.BlockSpec((RT, S), lambda i: (i, 0)),
            pl.BlockSpec((KT, DPAD), lambda i: (i, 0)),
        ],
        out_shape=(
            jax.ShapeDtypeStruct((T, K), jnp.float32),
            jax.ShapeDtypeStruct((T, S), jnp.int32),
            jax.ShapeDtypeStruct((T, S), jnp.float32),
            jax.ShapeDtypeStruct((K, DPAD), jnp.float32),
        ),
        scratch_shapes=[
            pltpu.VMEM((1, K), jnp.float32),
            pltpu.VMEM((K, G), jnp.float32),
            pltpu.VMEM((D, K), jnp.bfloat16),
        ],
        compiler_params=pltpu.CompilerParams(
            dimension_semantics=("arbitrary",),
            vmem_limit_bytes=100 * 1024 * 1024,
        ),
    )(x1, W)

    gathered = _sc_gather_fn()(wpad, idx8.reshape(T * S))

    out_flat, loss, ent = pl.pallas_call(
        _stage3_kernel,
        out_shape=(
            jax.ShapeDtypeStruct((T, D), jnp.float32),
            jax.ShapeDtypeStruct((1, 1), jnp.float32),
            jax.ShapeDtypeStruct((1, 1), jnp.float32),
        ),
    )(x1, gathered, prob8, idx8)

    return (loss[0, 0], out_flat.reshape(x0.shape), ent[0, 0], enc[None])


# speedup vs baseline: 15.2596x; 1.0854x over previous
import functools

import jax
import jax.numpy as jnp
from jax import lax
from jax.experimental import pallas as pl
from jax.experimental.pallas import tpu as pltpu
from jax.experimental.pallas import tpu_sc as plsc

T = 512
K = 8192
D = 400
G = 64
PG = 128
S = 8
COMMIT = 0.25

RT = 128
NSTEP = T // RT
NW = 32
BPW = (T * S) // NW

_HI = lax.Precision.HIGHEST


DPAD = 512
KT = K // NSTEP


def _stage1_kernel(x_ref, w_ref, enc_ref, idx_ref, prob_ref,
                   wpad_ref, ww_ref, amat_ref, wb_ref):
    step = pl.program_id(0)
    x = x_ref[...]
    w = w_ref[...]

    wpad_ref[:, :D] = w_ref[pl.ds(step * KT, KT), :]

    @pl.when(step == 0)
    def _precompute():
        ww_ref[...] = jnp.sum(w * w, axis=1)[None, :]
        a_row = lax.broadcasted_iota(jnp.int32, (K, G), 0) // PG
        a_col = lax.broadcasted_iota(jnp.int32, (K, G), 1)
        amat_ref[...] = (a_row == a_col).astype(jnp.float32)
        wb_ref[...] = w.astype(jnp.bfloat16).T

    xx = jnp.sum(x * x, axis=1, keepdims=True)
    xw = lax.dot_general(
        x.astype(jnp.bfloat16), wb_ref[...],
        (((1,), (0,)), ((), ())),
        preferred_element_type=jnp.float32)
    d = xx + ww_ref[...] - 2.0 * xw
    enc = 1.0 / d
    enc_ref[0] = enc

    d_group = lax.dot_general(
        d, amat_ref[...], (((1,), (0,)), ((), ())),
        precision=_HI, preferred_element_type=jnp.float32)
    gmin = jnp.min(d_group, axis=1, keepdims=True)
    giota = lax.broadcasted_iota(jnp.int32, (RT, G), 1)
    g = jnp.min(jnp.where(d_group == gmin, giota, G), axis=1, keepdims=True)

    cur = enc
    for b in (32, 16, 8, 4, 2, 1):
        half = cur.shape[1] // 2
        cur = jnp.where((g & b) != 0, cur[:, half:], cur[:, :half])
    e_win = cur

    lane = lax.broadcasted_iota(jnp.int32, (RT, PG), 1)
    cur = e_win
    ps = []
    js = []
    for _ in range(S):
        m = jnp.max(cur, axis=1, keepdims=True)
        j = jnp.min(jnp.where(cur == m, lane, PG), axis=1, keepdims=True)
        ps.append(m)
        js.append(j)
        cur = jnp.where(lane == j, -1.0, cur)

    denom = jnp.maximum(
        ps[0] + ps[1] + ps[2] + ps[3] + ps[4] + ps[5] + ps[6] + ps[7], 1e-12)
    prob_ref[...] = jnp.concatenate(ps, axis=1) / denom
    idx_ref[...] = (g * PG + jnp.concatenate(js, axis=1)).T


@functools.cache
def _sc_gather_fn():
    @functools.partial(
        pl.kernel,
        out_type=jax.ShapeDtypeStruct((T * S, DPAD), jnp.float32),
        mesh=plsc.VectorSubcoreMesh(core_axis_name="c", subcore_axis_name="s"),
        scratch_types=[
            pltpu.VMEM((BPW,), jnp.int32),
            pltpu.VMEM((BPW, DPAD), jnp.float32),
            pltpu.SemaphoreType.DMA,
        ],
    )
    def _sc_gather(w_hbm, idx_hbm, out_hbm, idx_v, rows_v, sem):
        wid = lax.axis_index("s") * 2 + lax.axis_index("c")
        base = wid * BPW
        pltpu.sync_copy(idx_hbm.at[pl.ds(base, BPW)], idx_v)
        pltpu.async_copy(w_hbm.at[idx_v], rows_v, sem).wait()
        pltpu.sync_copy(rows_v, out_hbm.at[pl.ds(base, BPW)])

    return _sc_gather


def _stage3_kernel(x_ref, gath_ref, prob_ref, idx_ref, out_ref, loss_ref,
                   ent_ref):
    x = x_ref[...]
    pr = prob_ref[...]
    g3 = gath_ref[...].reshape(S, T, DPAD)
    out = g3[0, :, :D] * pr[:, 0:1]
    for s in range(1, S):
        out = out + g3[s, :, :D] * pr[:, s:s + 1]
    out_ref[...] = (out - x) + x
    diff = x - out
    mse = jnp.sum(diff * diff, axis=(0, 1), keepdims=True) * (1.0 / (T * D))
    loss_ref[...] = mse + COMMIT * mse

    idx0 = idx_ref[...][0:1, :]
    eq = (idx0.reshape(T, 1) == idx0).astype(jnp.float32)
    n = jnp.sum(eq, axis=1, keepdims=True)
    ent_ref[...] = jnp.sum(
        jnp.log(n * (1.0 / T)), axis=(0, 1), keepdims=True) * (-1.0 / T)


def kernel(x0, W):
    x1 = x0.reshape(T, D)

    enc, idx8, prob8, wpad = pl.pallas_call(
        _stage1_kernel,
        grid=(NSTEP,),
        in_specs=[
            pl.BlockSpec((RT, D), lambda i: (i, 0)),
            pl.BlockSpec((K, D), lambda i: (0, 0)),
        ],
        out_specs=[
            pl.BlockSpec((1, RT, K), lambda i: (0, i, 0)),
            pl.BlockSpec((S, RT), lambda i: (0, i)),
            pl.BlockSpec((RT, S), lambda i: (i, 0)),
            pl.BlockSpec((KT, DPAD), lambda i: (i, 0)),
        ],
        out_shape=(
            jax.ShapeDtypeStruct((1, T, K), jnp.float32),
            jax.ShapeDtypeStruct((S, T), jnp.int32),
            jax.ShapeDtypeStruct((T, S), jnp.float32),
            jax.ShapeDtypeStruct((K, DPAD), jnp.float32),
        ),
        scratch_shapes=[
            pltpu.VMEM((1, K), jnp.float32),
            pltpu.VMEM((K, G), jnp.float32),
            pltpu.VMEM((D, K), jnp.bfloat16),
        ],
        compiler_params=pltpu.CompilerParams(
            dimension_semantics=("arbitrary",),
            vmem_limit_bytes=100 * 1024 * 1024,
        ),
    )(x1, W)

    gathered = _sc_gather_fn()(wpad, idx8.reshape(S * T))

    out_flat, loss, ent = pl.pallas_call(
        _stage3_kernel,
        out_shape=(
            jax.ShapeDtypeStruct((T, D), jnp.float32),
            jax.ShapeDtypeStruct((1, 1), jnp.float32),
            jax.ShapeDtypeStruct((1, 1), jnp.float32),
        ),
    )(x1, gathered, prob8, idx8)

    return (loss[0, 0], out_flat.reshape(x0.shape), ent[0, 0], enc)


# speedup vs baseline: 16.0031x; 1.0487x over previous
import functools

import jax
import jax.numpy as jnp
from jax import lax
from jax.experimental import pallas as pl
from jax.experimental.pallas import tpu as pltpu
from jax.experimental.pallas import tpu_sc as plsc

T = 512
K = 8192
D = 400
G = 64
PG = 128
S = 8
COMMIT = 0.25

RT = 128
NSTEP = T // RT
NW = 32
BPW = (T * S) // NW

_HI = lax.Precision.HIGHEST


DPAD = 512
KT = K // NSTEP


def _stage1_kernel(x_ref, w_ref, enc_ref, idx_ref, prob_ref,
                   wpad_ref, ww_ref, amat_ref, wb_ref):
    step = pl.program_id(0)
    x = x_ref[...]
    w = w_ref[...]

    wpad_ref[:, :D] = w_ref[pl.ds(step * KT, KT), :]

    @pl.when(step == 0)
    def _precompute():
        ww_ref[...] = jnp.sum(w * w, axis=1)[None, :]
        a_row = lax.broadcasted_iota(jnp.int32, (K, G), 0) // PG
        a_col = lax.broadcasted_iota(jnp.int32, (K, G), 1)
        amat_ref[...] = (a_row == a_col).astype(jnp.float32)
        wb_ref[...] = w.astype(jnp.bfloat16).T

    xx = jnp.sum(x * x, axis=1, keepdims=True)
    xw = lax.dot_general(
        x.astype(jnp.bfloat16), wb_ref[...],
        (((1,), (0,)), ((), ())),
        preferred_element_type=jnp.float32)
    d = xx + ww_ref[...] - 2.0 * xw
    enc = 1.0 / d
    enc_ref[0] = enc

    d_group = lax.dot_general(
        d, amat_ref[...], (((1,), (0,)), ((), ())),
        precision=_HI, preferred_element_type=jnp.float32)
    gmin = jnp.min(d_group, axis=1, keepdims=True)
    giota = lax.broadcasted_iota(jnp.int32, (RT, G), 1)
    g = jnp.min(jnp.where(d_group == gmin, giota, G), axis=1, keepdims=True)

    cur = enc
    for b in (32, 16, 8, 4, 2, 1):
        half = cur.shape[1] // 2
        cur = jnp.where((g & b) != 0, cur[:, half:], cur[:, :half])
    e_win = cur

    lane = lax.broadcasted_iota(jnp.int32, (RT, PG), 1)
    cur = e_win
    ps = []
    js = []
    for _ in range(S):
        m = jnp.max(cur, axis=1, keepdims=True)
        j = jnp.min(jnp.where(cur == m, lane, PG), axis=1, keepdims=True)
        ps.append(m)
        js.append(j)
        cur = jnp.where(lane == j, -1.0, cur)

    denom = jnp.maximum(
        ps[0] + ps[1] + ps[2] + ps[3] + ps[4] + ps[5] + ps[6] + ps[7], 1e-12)
    prob_ref[...] = jnp.concatenate(ps, axis=1) / denom
    idx_ref[...] = (g * PG + jnp.concatenate(js, axis=1)).T


@functools.cache
def _sc_gather_fn():
    @functools.partial(
        pl.kernel,
        out_type=jax.ShapeDtypeStruct((T * S, DPAD), jnp.float32),
        mesh=plsc.VectorSubcoreMesh(core_axis_name="c", subcore_axis_name="s"),
        scratch_types=[
            pltpu.VMEM((BPW,), jnp.int32),
            pltpu.VMEM((BPW, DPAD), jnp.float32),
            pltpu.SemaphoreType.DMA,
        ],
    )
    def _sc_gather(w_hbm, idx_hbm, out_hbm, idx_v, rows_v, sem):
        wid = lax.axis_index("s") * 2 + lax.axis_index("c")
        s_row = wid // (T // BPW)
        t0 = (wid % (T // BPW)) * BPW
        pltpu.sync_copy(idx_hbm.at[s_row, pl.ds(t0, BPW)], idx_v)
        pltpu.async_copy(w_hbm.at[idx_v], rows_v, sem).wait()
        pltpu.sync_copy(rows_v, out_hbm.at[pl.ds(wid * BPW, BPW)])

    return _sc_gather


def _stage3_kernel(x_ref, gath_ref, prob_ref, idx_ref, out_ref, loss_ref,
                   ent_ref):
    x = x_ref[...]
    pr = prob_ref[...]
    g3 = gath_ref[...].reshape(S, T, DPAD)
    out = g3[0, :, :D] * pr[:, 0:1]
    for s in range(1, S):
        out = out + g3[s, :, :D] * pr[:, s:s + 1]
    out_ref[...] = ((out - x) + x).reshape(8, T // 8, 1, D)
    diff = x - out
    mse = jnp.sum(diff * diff, axis=(0, 1), keepdims=True) * (1.0 / (T * D))
    loss_ref[...] = mse + COMMIT * mse

    idx0 = idx_ref[...][0:1, :]
    eq = (idx0.reshape(T, 1) == idx0).astype(jnp.float32)
    n = jnp.sum(eq, axis=1, keepdims=True)
    ent_ref[...] = jnp.sum(
        jnp.log(n * (1.0 / T)), axis=(0, 1), keepdims=True) * (-1.0 / T)


def kernel(x0, W):
    x1 = x0.reshape(T, D)

    enc, idx8, prob8, wpad = pl.pallas_call(
        _stage1_kernel,
        grid=(NSTEP,),
        in_specs=[
            pl.BlockSpec((RT, D), lambda i: (i, 0)),
            pl.BlockSpec((K, D), lambda i: (0, 0)),
        ],
        out_specs=[
            pl.BlockSpec((1, RT, K), lambda i: (0, i, 0)),
            pl.BlockSpec((S, RT), lambda i: (0, i)),
            pl.BlockSpec((RT, S), lambda i: (i, 0)),
            pl.BlockSpec((KT, DPAD), lambda i: (i, 0)),
        ],
        out_shape=(
            jax.ShapeDtypeStruct((1, T, K), jnp.float32),
            jax.ShapeDtypeStruct((S, T), jnp.int32),
            jax.ShapeDtypeStruct((T, S), jnp.float32),
            jax.ShapeDtypeStruct((K, DPAD), jnp.float32),
        ),
        scratch_shapes=[
            pltpu.VMEM((1, K), jnp.float32),
            pltpu.VMEM((K, G), jnp.float32),
            pltpu.VMEM((D, K), jnp.bfloat16),
        ],
        compiler_params=pltpu.CompilerParams(
            dimension_semantics=("arbitrary",),
            vmem_limit_bytes=100 * 1024 * 1024,
        ),
    )(x1, W)

    gathered = _sc_gather_fn()(wpad, idx8)

    out0, loss, ent = pl.pallas_call(
        _stage3_kernel,
        out_shape=(
            jax.ShapeDtypeStruct((8, T // 8, 1, D), jnp.float32),
            jax.ShapeDtypeStruct((1, 1), jnp.float32),
            jax.ShapeDtypeStruct((1, 1), jnp.float32),
        ),
    )(x1, gathered, prob8, idx8)

    return (loss[0, 0], out0, ent[0, 0], enc)


# speedup vs baseline: 16.0271x; 1.0015x over previous
import functools

import jax
import jax.numpy as jnp
from jax import lax
from jax.experimental import pallas as pl
from jax.experimental.pallas import tpu as pltpu
from jax.experimental.pallas import tpu_sc as plsc

T = 512
K = 8192
D = 400
G = 64
PG = 128
S = 8
COMMIT = 0.25

RT = 128
NSTEP = T // RT
NW = 32
BPW = (T * S) // NW

_HI = lax.Precision.HIGHEST


DPAD = 512
KT = K // NSTEP


def _stage1_kernel(x_ref, w_ref, enc_ref, idx_ref, prob_ref,
                   wpad_ref, ww_ref, amat_ref, wb_ref):
    step = pl.program_id(0)
    x = x_ref[...]
    w = w_ref[...]

    wpad_ref[:, :D] = w_ref[pl.ds(step * KT, KT), :]

    @pl.when(step == 0)
    def _precompute():
        ww_ref[...] = jnp.sum(w * w, axis=1)[None, :]
        a_row = lax.broadcasted_iota(jnp.int32, (K, G), 0) // PG
        a_col = lax.broadcasted_iota(jnp.int32, (K, G), 1)
        amat_ref[...] = (a_row == a_col).astype(jnp.float32)
        wb_ref[...] = w.astype(jnp.bfloat16).T

    xx = jnp.sum(x * x, axis=1, keepdims=True)
    xw = lax.dot_general(
        x.astype(jnp.bfloat16), wb_ref[...],
        (((1,), (0,)), ((), ())),
        preferred_element_type=jnp.float32)
    d = xx + ww_ref[...] - 2.0 * xw
    enc = 1.0 / d
    enc_ref[0] = enc

    d_group = lax.dot_general(
        d, amat_ref[...], (((1,), (0,)), ((), ())),
        precision=_HI, preferred_element_type=jnp.float32)
    gmin = jnp.min(d_group, axis=1, keepdims=True)
    giota = lax.broadcasted_iota(jnp.int32, (RT, G), 1)
    g = jnp.min(jnp.where(d_group == gmin, giota, G), axis=1, keepdims=True)

    cur = enc
    for b in (32, 16, 8, 4, 2, 1):
        half = cur.shape[1] // 2
        cur = jnp.where((g & b) != 0, cur[:, half:], cur[:, :half])
    e_win = cur

    lane = lax.broadcasted_iota(jnp.int32, (RT, PG), 1)
    cur = e_win
    ps = []
    js = []
    for _ in range(S):
        m = jnp.max(cur, axis=1, keepdims=True)
        j = jnp.min(jnp.where(cur == m, lane, PG), axis=1, keepdims=True)
        ps.append(m)
        js.append(j)
        cur = jnp.where(lane == j, -1.0, cur)

    denom = jnp.maximum(
        ps[0] + ps[1] + ps[2] + ps[3] + ps[4] + ps[5] + ps[6] + ps[7], 1e-12)
    prob_ref[...] = jnp.concatenate(ps, axis=1) / denom
    idx_ref[...] = (g * PG + jnp.concatenate(js, axis=1)).T


@functools.cache
def _sc_gather_fn():
    @functools.partial(
        pl.kernel,
        out_type=jax.ShapeDtypeStruct((T * S, DPAD), jnp.float32),
        mesh=plsc.VectorSubcoreMesh(core_axis_name="c", subcore_axis_name="s"),
        scratch_types=[
            pltpu.VMEM((BPW,), jnp.int32),
            pltpu.VMEM((BPW, DPAD), jnp.float32),
            pltpu.SemaphoreType.DMA,
        ],
        compiler_params=pltpu.CompilerParams(use_tc_tiling_on_sc=True),
    )
    def _sc_gather(w_hbm, idx_hbm, out_hbm, idx_v, rows_v, sem):
        wid = lax.axis_index("s") * 2 + lax.axis_index("c")
        s_row = wid // (T // BPW)
        t0 = (wid % (T // BPW)) * BPW
        pltpu.sync_copy(idx_hbm.at[s_row, pl.ds(t0, BPW)], idx_v)
        pltpu.async_copy(w_hbm.at[idx_v], rows_v, sem).wait()
        pltpu.sync_copy(rows_v, out_hbm.at[pl.ds(wid * BPW, BPW)])

    return _sc_gather


def _stage3_kernel(x_ref, gath_ref, prob_ref, idx_ref, out_ref, loss_ref,
                   ent_ref):
    x = x_ref[...]
    pr = prob_ref[...]
    g3 = gath_ref[...].reshape(S, T, DPAD)
    out = g3[0, :, :D] * pr[:, 0:1]
    for s in range(1, S):
        out = out + g3[s, :, :D] * pr[:, s:s + 1]
    out_ref[...] = ((out - x) + x).reshape(8, T // 8, 1, D)
    diff = x - out
    mse = jnp.sum(diff * diff, axis=(0, 1), keepdims=True) * (1.0 / (T * D))
    loss_ref[...] = mse + COMMIT * mse

    idx0 = idx_ref[...][0:1, :]
    eq = (idx0.reshape(T, 1) == idx0).astype(jnp.float32)
    n = jnp.sum(eq, axis=1, keepdims=True)
    ent_ref[...] = jnp.sum(
        jnp.log(n * (1.0 / T)), axis=(0, 1), keepdims=True) * (-1.0 / T)


def kernel(x0, W):
    x1 = x0.reshape(T, D)

    enc, idx8, prob8, wpad = pl.pallas_call(
        _stage1_kernel,
        grid=(NSTEP,),
        in_specs=[
            pl.BlockSpec((RT, D), lambda i: (i, 0)),
            pl.BlockSpec((K, D), lambda i: (0, 0)),
        ],
        out_specs=[
            pl.BlockSpec((1, RT, K), lambda i: (0, i, 0)),
            pl.BlockSpec((S, RT), lambda i: (0, i)),
            pl.BlockSpec((RT, S), lambda i: (i, 0)),
            pl.BlockSpec((KT, DPAD), lambda i: (i, 0)),
        ],
        out_shape=(
            jax.ShapeDtypeStruct((1, T, K), jnp.float32),
            jax.ShapeDtypeStruct((S, T), jnp.int32),
            jax.ShapeDtypeStruct((T, S), jnp.float32),
            jax.ShapeDtypeStruct((K, DPAD), jnp.float32),
        ),
        scratch_shapes=[
            pltpu.VMEM((1, K), jnp.float32),
            pltpu.VMEM((K, G), jnp.float32),
            pltpu.VMEM((D, K), jnp.bfloat16),
        ],
        compiler_params=pltpu.CompilerParams(
            dimension_semantics=("arbitrary",),
            vmem_limit_bytes=100 * 1024 * 1024,
        ),
    )(x1, W)

    gathered = _sc_gather_fn()(wpad, idx8)

    out0, loss, ent = pl.pallas_call(
        _stage3_kernel,
        out_shape=(
            jax.ShapeDtypeStruct((8, T // 8, 1, D), jnp.float32),
            jax.ShapeDtypeStruct((1, 1), jnp.float32),
            jax.ShapeDtypeStruct((1, 1), jnp.float32),
        ),
    )(x1, gathered, prob8, idx8)

    return (loss[0, 0], out0, ent[0, 0], enc)
